# baseline (speedup 1.0000x reference)
import functools

import jax
import jax.numpy as jnp
from jax import lax
from jax.experimental import pallas as pl
from jax.experimental.pallas import tpu as pltpu

_N_PASS = 8

_SORT8_NET = (
    (0, 1), (2, 3), (4, 5), (6, 7),
    (0, 2), (1, 3), (4, 6), (5, 7),
    (1, 2), (5, 6),
    (0, 4), (1, 5), (2, 6), (3, 7),
    (2, 4), (3, 5),
    (1, 2), (3, 4), (5, 6),
)

_BITONIC8_NET = (
    (0, 4), (1, 5), (2, 6), (3, 7),
    (0, 2), (1, 3), (4, 6), (5, 7),
    (0, 1), (2, 3), (4, 5), (6, 7),
)


def _ce(v, i, j):
    hi = jnp.maximum(v[i], v[j])
    lo = jnp.minimum(v[i], v[j])
    v[i], v[j] = hi, lo


def _sort8(v):
    for i, j in _SORT8_NET:
        _ce(v, i, j)
    return v


def _merge_top8(r, s):
    t = [jnp.maximum(r[i], s[7 - i]) for i in range(8)]
    for i, j in _BITONIC8_NET:
        _ce(t, i, j)
    return t


def _lowest_passing(raw_ref, row0, rows, C):
    def group_vals(c0):
        x = raw_ref[0, pl.ds(c0, 8), pl.ds(row0, rows), :]
        return _sort8([x[k] for k in range(8)])

    if C >= 16:
        n_pairs = C // 16

        def step(g, banks):
            ra, rb = banks
            sa = group_vals(g * 16)
            sb = group_vals(g * 16 + 8)
            return (
                tuple(_merge_top8(list(ra), sa)),
                tuple(_merge_top8(list(rb), sb)),
            )

        banks = (tuple(group_vals(0)), tuple(group_vals(8)))
        if n_pairs > 1:
            banks = lax.fori_loop(1, n_pairs, step, banks)
        regs = _merge_top8(list(banks[0]), list(banks[1]))
        done = n_pairs * 16
    else:
        regs = group_vals(0)
        done = 8

    for c in range(done, C):
        carry = raw_ref[0, c, pl.ds(row0, rows), :]
        for k in range(_N_PASS):
            r = regs[k]
            regs[k] = jnp.maximum(r, carry)
            if k + 1 < _N_PASS:
                carry = jnp.minimum(r, carry)
    return regs[_N_PASS - 1]


def _pool_body(enc_ref, raw_ref, out_ref, *, C, tho, W):
    L = W // 2
    Lo = L // 2
    i32 = jnp.int32

    quad_rows = min(8, 4 * tho)
    out_rows_per_quad = quad_rows // 4
    n_quads = (4 * tho) // quad_rows

    perm1 = jnp.concatenate(
        [jnp.arange(0, L, 2, dtype=i32), jnp.arange(1, L, 2, dtype=i32)]
    )[None, :]
    dup1 = (jnp.arange(L, dtype=i32) // 2)[None, :]
    lane1 = jnp.arange(L, dtype=i32)[None, :]

    def lane_gather(x, idx):
        return jnp.take_along_axis(
            x,
            jnp.broadcast_to(idx, x.shape),
            axis=1,
            mode="promise_in_bounds",
        )

    for q in range(n_quads):
        row0 = q * quad_rows
        lp = _lowest_passing(raw_ref, row0, quad_rows, C)
        p = lane_gather(lp, perm1)
        lp0, lp1 = p[:, :Lo], p[:, Lo:]

        top_exps, lane_idxs = [], []
        for k in range(2):
            tops, dws = [], []
            for rr in range(out_rows_per_quad):
                ra = 4 * rr + k
                rb = ra + 2
                a0 = lp0[ra : ra + 1, :]
                a1 = lp1[ra : ra + 1, :]
                b0 = lp0[rb : rb + 1, :]
                b1 = lp1[rb : rb + 1, :]
                m = jnp.maximum(jnp.maximum(a0, a1), jnp.maximum(b0, b1))
                top = (a0 == m) | (a1 == m)
                dw = 1 - (jnp.where(top, a0, b0) == m).astype(i32)
                tops.append(top.astype(i32))
                dws.append(dw)
            if out_rows_per_quad > 1:
                top_s = jnp.concatenate(tops, axis=0)
                dw_s = jnp.concatenate(dws, axis=0)
            else:
                top_s, dw_s = tops[0], dws[0]
            pad = jnp.zeros_like(top_s)
            top_exps.append(
                lane_gather(jnp.concatenate([top_s, pad], axis=1), dup1)
            )
            dw_pad = jnp.concatenate([dw_s, pad], axis=1)
            lane_idxs.append(jnp.where(lane1 < Lo, 2 * lane1 + dw_pad, 0))

        for rr in range(out_rows_per_quad):
            orow = q * out_rows_per_quad + rr
            ea = enc_ref[0, :, 2 * orow, :]
            eb = enc_ref[0, :, 2 * orow + 1, :]
            for k in range(2):
                eak = ea[:, k * L : (k + 1) * L]
                ebk = eb[:, k * L : (k + 1) * L]
                g = jnp.where(top_exps[k][rr : rr + 1, :] > 0, eak, ebk)
                out = lane_gather(g, lane_idxs[k][rr : rr + 1, :])
                out_ref[0, :, orow, pl.ds(k * Lo, Lo)] = out[:, :Lo]


def kernel(encoded, raw_activations):
    B, C, H, W = encoded.shape
    if H % 2 or W % 2:
        encoded = jnp.pad(encoded, ((0, 0), (0, 0), (0, H % 2), (0, W % 2)))
        raw_activations = jnp.pad(
            raw_activations, ((0, 0), (0, 0), (0, H % 2), (0, W % 2))
        )
        H += H % 2
        W += W % 2
    Ho, Wo = H // 2, W // 2

    if W % 4:
        encoded = jnp.pad(encoded, ((0, 0), (0, 0), (0, 0), (0, 2)))
        raw_activations = jnp.pad(
            raw_activations,
            ((0, 0), (0, 0), (0, 0), (0, 2)),
            constant_values=-jnp.inf,
        )
        W += 2

    raw_v = raw_activations.reshape(B, C, 2 * H, W // 2)

    tho = 8
    while Ho % tho:
        tho //= 2

    body = functools.partial(_pool_body, C=C, tho=tho, W=W)
    out = pl.pallas_call(
        body,
        grid=(B, Ho // tho),
        in_specs=[
            pl.BlockSpec((1, C, 2 * tho, W), lambda b, j: (b, 0, j, 0)),
            pl.BlockSpec((1, C, 4 * tho, W // 2), lambda b, j: (b, 0, j, 0)),
        ],
        out_specs=pl.BlockSpec((1, C, tho, W // 2), lambda b, j: (b, 0, j, 0)),
        out_shape=jax.ShapeDtypeStruct((B, C, Ho, W // 2), jnp.float32),
        compiler_params=pltpu.CompilerParams(
            dimension_semantics=("parallel", "parallel"),
        ),
    )(encoded, raw_v)
    return out[:, :, :, :Wo]


# speedup vs baseline: 1.0802x; 1.0802x over previous
import functools

import jax
import jax.numpy as jnp
from jax import lax
from jax.experimental import pallas as pl
from jax.experimental.pallas import tpu as pltpu

_N_PASS = 8

_SORT8_NET = (
    (0, 1), (2, 3), (4, 5), (6, 7),
    (0, 2), (1, 3), (4, 6), (5, 7),
    (1, 2), (5, 6),
    (0, 4), (1, 5), (2, 6), (3, 7),
    (2, 4), (3, 5),
    (1, 2), (3, 4), (5, 6),
)

_BITONIC8_NET = (
    (0, 4), (1, 5), (2, 6), (3, 7),
    (0, 2), (1, 3), (4, 6), (5, 7),
    (0, 1), (2, 3), (4, 5), (6, 7),
)


def _ce(v, i, j):
    hi = jnp.maximum(v[i], v[j])
    lo = jnp.minimum(v[i], v[j])
    v[i], v[j] = hi, lo


def _sort8(v):
    for i, j in _SORT8_NET:
        _ce(v, i, j)
    return v


def _merge_top8(r, s):
    t = [jnp.maximum(r[i], s[7 - i]) for i in range(8)]
    for i, j in _BITONIC8_NET:
        _ce(t, i, j)
    return t


def _lowest_passing(raw_ref, row0, rows, C):
    def group_vals(c0):
        x = raw_ref[0, pl.ds(c0, 8), pl.ds(row0, rows), :]
        return _sort8([x[k] for k in range(8)])

    if C >= 24:
        n_trips = C // 24

        def step(g, banks):
            ra, rb, rc = banks
            sa = group_vals(g * 24)
            sb = group_vals(g * 24 + 8)
            sc = group_vals(g * 24 + 16)
            return (
                tuple(_merge_top8(list(ra), sa)),
                tuple(_merge_top8(list(rb), sb)),
                tuple(_merge_top8(list(rc), sc)),
            )

        banks = (tuple(group_vals(0)), tuple(group_vals(8)), tuple(group_vals(16)))
        if n_trips > 1:
            banks = lax.fori_loop(1, n_trips, step, banks)
        regs = _merge_top8(
            _merge_top8(list(banks[0]), list(banks[1])), list(banks[2])
        )
        done = n_trips * 24
    elif C >= 16:
        regs = _merge_top8(list(group_vals(0)), group_vals(8))
        done = 16
    else:
        regs = group_vals(0)
        done = 8

    for c in range(done, C):
        carry = raw_ref[0, c, pl.ds(row0, rows), :]
        for k in range(_N_PASS):
            r = regs[k]
            regs[k] = jnp.maximum(r, carry)
            if k + 1 < _N_PASS:
                carry = jnp.minimum(r, carry)
    return regs[_N_PASS - 1]


def _pool_body(enc_ref, raw_ref, out_ref, *, C, n_out_rows, L):
    Lo = L // 2
    i32 = jnp.int32

    half_rows = 8 if n_out_rows >= 4 else 2 * n_out_rows
    out_rows_per_half = half_rows // 2
    n_halves = (2 * n_out_rows) // half_rows

    perm1 = jnp.concatenate(
        [jnp.arange(0, L, 2, dtype=i32), jnp.arange(1, L, 2, dtype=i32)]
    )[None, :]
    dup1 = (jnp.arange(L, dtype=i32) // 2)[None, :]
    lane1 = jnp.arange(L, dtype=i32)[None, :]

    def lane_gather(x, idx):
        return jnp.take_along_axis(
            x,
            jnp.broadcast_to(idx, x.shape),
            axis=1,
            mode="promise_in_bounds",
        )

    for h in range(n_halves):
        row0 = h * half_rows
        lp = _lowest_passing(raw_ref, row0, half_rows, C)
        p = lane_gather(lp, perm1)
        lp0, lp1 = p[:, :Lo], p[:, Lo:]

        tops, dws = [], []
        for r in range(out_rows_per_half):
            ra = 4 * (r // 2) + (r % 2)
            rb = ra + 2
            a0 = lp0[ra : ra + 1, :]
            a1 = lp1[ra : ra + 1, :]
            b0 = lp0[rb : rb + 1, :]
            b1 = lp1[rb : rb + 1, :]
            m = jnp.maximum(jnp.maximum(a0, a1), jnp.maximum(b0, b1))
            top = (a0 == m) | (a1 == m)
            dw = 1 - (jnp.where(top, a0, b0) == m).astype(i32)
            tops.append(top.astype(i32))
            dws.append(dw)
        top_s = jnp.concatenate(tops, axis=0)
        dw_s = jnp.concatenate(dws, axis=0)
        pad = jnp.zeros_like(top_s)
        top_exp = lane_gather(jnp.concatenate([top_s, pad], axis=1), dup1)
        dw_pad = jnp.concatenate([dw_s, pad], axis=1)
        lane_idx = jnp.where(lane1 < Lo, 2 * lane1 + dw_pad, 0)

        for r in range(out_rows_per_half):
            ra = row0 + 4 * (r // 2) + (r % 2)
            ea = enc_ref[0, :, ra, :]
            eb = enc_ref[0, :, ra + 2, :]
            g = jnp.where(top_exp[r : r + 1, :] > 0, ea, eb)
            out = lane_gather(g, lane_idx[r : r + 1, :])
            out_ref[0, :, h * out_rows_per_half + r, :] = out[:, :Lo]


def kernel(encoded, raw_activations):
    B, C, H, W = encoded.shape
    if H % 2 or W % 2:
        encoded = jnp.pad(encoded, ((0, 0), (0, 0), (0, H % 2), (0, W % 2)))
        raw_activations = jnp.pad(
            raw_activations, ((0, 0), (0, 0), (0, H % 2), (0, W % 2))
        )
        H += H % 2
        W += W % 2
    Ho, Wo = H // 2, W // 2

    if W % 4:
        encoded = jnp.pad(encoded, ((0, 0), (0, 0), (0, 0), (0, 2)))
        raw_activations = jnp.pad(
            raw_activations,
            ((0, 0), (0, 0), (0, 0), (0, 2)),
            constant_values=-jnp.inf,
        )
        W += 2

    L = W // 2
    enc_v = encoded.reshape(B, C, 2 * H, L)
    raw_v = raw_activations.reshape(B, C, 2 * H, L)

    n_out_rows = 8
    while (2 * Ho) % n_out_rows:
        n_out_rows //= 2

    body = functools.partial(_pool_body, C=C, n_out_rows=n_out_rows, L=L)
    out = pl.pallas_call(
        body,
        grid=(B, (2 * Ho) // n_out_rows),
        in_specs=[
            pl.BlockSpec((1, C, 2 * n_out_rows, L), lambda b, j: (b, 0, j, 0)),
            pl.BlockSpec((1, C, 2 * n_out_rows, L), lambda b, j: (b, 0, j, 0)),
        ],
        out_specs=pl.BlockSpec((1, C, n_out_rows, L // 2), lambda b, j: (b, 0, j, 0)),
        out_shape=jax.ShapeDtypeStruct((B, C, 2 * Ho, L // 2), jnp.float32),
        compiler_params=pltpu.CompilerParams(
            dimension_semantics=("parallel", "parallel"),
        ),
    )(enc_v, raw_v)
    out = out.reshape(B, C, Ho, W // 2)
    return out[:, :, :, :Wo]
